# baseline (speedup 1.0000x reference)
import functools

import jax
import jax.numpy as jnp
from jax import lax
from jax.experimental import pallas as pl
from jax.experimental.pallas import tpu as pltpu
from jax.experimental.pallas import tpu_sc as plsc

_N = 10000
_E = 320000
_D = 128
_DE = 16
_H = 256
_C = 10
_G = 64

_NC = 2
_NS = 16

_EB = 2000
_NB = 1000
_NGB = _N // _NB
_CB = 40
_ET1 = _E // (_NC * _NS)
_ET2 = _E // _NS
_NP = 10240
_ZR = _NP // _NS
_ZCH = 16


def _enc_p1_body(ea_ref, we1_ref, be1_ref, we2_ref, be2_ref, wc_ref, bc_ref,
                 p1_ref, wf_ref, bf_ref):
    @pl.when(pl.program_id(0) == 0)
    def _():
        wf_ref[...] = jnp.dot(we2_ref[...], wc_ref[...],
                              preferred_element_type=jnp.float32)
        bf_ref[...] = (
            jnp.dot(be2_ref[...], wc_ref[...],
                    preferred_element_type=jnp.float32) + bc_ref[...])

    t = jnp.maximum(
        jnp.dot(ea_ref[...], we1_ref[...], preferred_element_type=jnp.float32)
        + be1_ref[...], 0.0)
    p1_ref[...] = (
        jnp.dot(t, wf_ref[...], preferred_element_type=jnp.float32)
        + bf_ref[...])


def _enc_p1(edge_attr, we1, be1, we2, be2, wc, bc):
    return pl.pallas_call(
        _enc_p1_body,
        grid=(_E // _EB,),
        in_specs=[
            pl.BlockSpec((_EB, _DE), lambda i: (i, 0)),
            pl.BlockSpec((_DE, _H), lambda i: (0, 0)),
            pl.BlockSpec((1, _H), lambda i: (0, 0)),
            pl.BlockSpec((_H, _H), lambda i: (0, 0)),
            pl.BlockSpec((1, _H), lambda i: (0, 0)),
            pl.BlockSpec((_H, _D), lambda i: (0, 0)),
            pl.BlockSpec((1, _D), lambda i: (0, 0)),
        ],
        out_specs=pl.BlockSpec((_EB, _D), lambda i: (i, 0)),
        out_shape=jax.ShapeDtypeStruct((_E, _D), jnp.float32),
        scratch_shapes=[
            pltpu.VMEM((_H, _D), jnp.float32),
            pltpu.VMEM((1, _D), jnp.float32),
        ],
    )(edge_attr, we1, be1, we2, be2, wc, bc)


def _enc_ea_body(ea_ref, we1_ref, be1_ref, we2_ref, be2_ref, eac_ref):
    t = jnp.maximum(
        jnp.dot(ea_ref[...], we1_ref[...], preferred_element_type=jnp.float32)
        + be1_ref[...], 0.0)
    ea = jnp.dot(t, we2_ref[...], preferred_element_type=jnp.float32) + be2_ref[...]
    eac_ref[0] = ea[:, :_D]
    eac_ref[1] = ea[:, _D:]


def _enc_ea(edge_attr, we1, be1, we2, be2):
    return pl.pallas_call(
        _enc_ea_body,
        grid=(_E // _EB,),
        in_specs=[
            pl.BlockSpec((_EB, _DE), lambda i: (i, 0)),
            pl.BlockSpec((_DE, _H), lambda i: (0, 0)),
            pl.BlockSpec((1, _H), lambda i: (0, 0)),
            pl.BlockSpec((_H, _H), lambda i: (0, 0)),
            pl.BlockSpec((1, _H), lambda i: (0, 0)),
        ],
        out_specs=pl.BlockSpec((2, _EB, _D), lambda i: (0, i, 0)),
        out_shape=jax.ShapeDtypeStruct((2, _E, _D), jnp.float32),
    )(edge_attr, we1, be1, we2, be2)


def _zero_acc(zb_v, acc_sh, sid):
    @pl.loop(0, _ZCH)
    def _(j):
        for k in range(_D // 16):
            zb_v[j, pl.ds(k * 16, 16)] = jnp.zeros((16,), jnp.float32)

    @pl.loop(0, _ZR // _ZCH)
    def _(k):
        pltpu.sync_copy(zb_v, acc_sh.at[pl.ds(sid * _ZR + k * _ZCH, _ZCH)])


def _mp_pipeline(tab_hbm, lin3_hbm, srcA_hbm, srcB_hbm, dst_hbm, acc_sh,
                 srcv, dstv, xg, pv, sg, sl, ssrc, sdst, ss,
                 *, nch, ebase, lin_row, cid):
    def issue_idx(i, b2, b3):
        off = ebase + i * _CB

        @pl.when(cid == 0)
        def _():
            pltpu.async_copy(srcA_hbm.at[pl.ds(off, _CB)], srcv.at[b2],
                             ssrc[b2])

        @pl.when(cid == 1)
        def _():
            pltpu.async_copy(srcB_hbm.at[pl.ds(off, _CB)], srcv.at[b2],
                             ssrc[b2])

        pltpu.async_copy(dst_hbm.at[pl.ds(off, _CB)], dstv.at[b3], sdst[b3])

    def wait_src(b2):
        pltpu.make_async_copy(
            srcA_hbm.at[pl.ds(ebase, _CB)], srcv.at[b2], ssrc[b2]).wait()

    def wait_dst(b3):
        pltpu.make_async_copy(
            dst_hbm.at[pl.ds(ebase, _CB)], dstv.at[b3], sdst[b3]).wait()

    def issue_in(i, b2, b3):
        pltpu.async_copy(tab_hbm.at[srcv.at[b2]], xg[b2], sg[b2])
        pltpu.async_copy(lin3_hbm.at[lin_row, pl.ds(ebase + i * _CB, _CB)],
                         pv[b3], sl[b3])

    def wait_in(b2, b3):
        pltpu.make_async_copy(tab_hbm.at[srcv.at[b2]], xg[b2], sg[b2]).wait()
        pltpu.make_async_copy(lin3_hbm.at[lin_row, pl.ds(ebase, _CB)],
                              pv[b3], sl[b3]).wait()

    def compute(b2, b3):
        xb, pb = xg[b2], pv[b3]

        @pl.loop(0, _CB)
        def _(j):
            for k in range(_D // 16):
                sli = pl.ds(k * 16, 16)
                pb[j, sli] = jnp.maximum(xb[j, sli] + pb[j, sli], 0.0)

    def issue_sc(b3):
        pltpu.async_copy(pv[b3], acc_sh.at[dstv.at[b3]], ss[b3], add=True)

    def wait_ss(b3):
        pltpu.make_async_copy(pv[b3], acc_sh.at[dstv.at[b3]], ss[b3]).wait()

    def step(i, b2, b3, first=False, issue1=True, issue2=True):
        wait_in(b2, b3)
        if issue1:
            wait_src(1 - b2)
            issue_in(i + 1, 1 - b2, (b3 + 1) % 3)
        compute(b2, b3)
        wait_dst(b3)
        issue_sc(b3)
        if not first:
            wait_ss((b3 + 2) % 3)
        if issue2:
            issue_idx(i + 2, b2, (b3 + 2) % 3)

    issue_idx(0, 0, 0)
    issue_idx(1, 1, 1)
    wait_src(0)
    issue_in(0, 0, 0)
    step(0, 0, 0, first=True)
    step(1, 1, 1)

    steady = (nch - 4) // 6

    @pl.loop(0, steady)
    def _(s):
        i0 = 2 + s * 6
        for u in range(6):
            step(i0 + u, (2 + u) % 2, (2 + u) % 3)

    for i in range(2 + 6 * steady, nch):
        step(i, i % 2, i % 3, issue1=(i + 1 < nch), issue2=(i + 2 < nch))
    wait_ss((nch - 1) % 3)


def _flush_acc(acc_sh, out_hbm, cid, sid):
    row0 = sid * _ZR
    pltpu.sync_copy(acc_sh.at[pl.ds(row0, _ZR)],
                    out_hbm.at[cid, pl.ds(row0, _ZR)])


_SC_SCRATCH = [
    pltpu.VMEM((2, _CB), jnp.int32),
    pltpu.VMEM((3, _CB), jnp.int32),
    pltpu.VMEM((_CB, _D), jnp.float32),
    pltpu.VMEM((_CB, _D), jnp.float32),
    pltpu.VMEM((_CB, _D), jnp.float32),
    pltpu.VMEM((_CB, _D), jnp.float32),
    pltpu.VMEM((_CB, _D), jnp.float32),
    pltpu.VMEM((_ZCH, _D), jnp.float32),
    pltpu.VMEM_SHARED((_NP, _D), jnp.float32),
] + [pltpu.SemaphoreType.DMA] * 13


def _b1_body(x_hbm, src_hbm, dst_hbm, p13_hbm, out_hbm,
             srcv, dstv, xg0, xg1, pv0, pv1, pv2, zb_v, acc_sh, *sems):
    cid = lax.axis_index("c")
    sid = lax.axis_index("s")
    wid = cid * _NS + sid
    _zero_acc(zb_v, acc_sh, sid)
    plsc.subcore_barrier()
    _mp_pipeline(x_hbm, p13_hbm, src_hbm, src_hbm, dst_hbm, acc_sh,
                 srcv, dstv, (xg0, xg1), (pv0, pv1, pv2),
                 sems[0:2], sems[2:5], sems[5:7], sems[7:10], sems[10:13],
                 nch=_ET1 // _CB, ebase=wid * _ET1, lin_row=0, cid=cid)
    plsc.subcore_barrier()
    _flush_acc(acc_sh, out_hbm, cid, sid)


def _b1(x, src, dst, p13):
    return pl.kernel(
        _b1_body,
        out_type=jax.ShapeDtypeStruct((_NC, _NP, _D), jnp.float32),
        mesh=plsc.VectorSubcoreMesh(core_axis_name="c", subcore_axis_name="s"),
        scratch_types=_SC_SCRATCH,
    )(x, src, dst, p13)


def _b2_body(h_hbm, src_hbm, srchi_hbm, dst_hbm, ea3_hbm, out_hbm,
             srcv, dstv, xg0, xg1, pv0, pv1, pv2, zb_v, acc_sh, *sems):
    cid = lax.axis_index("c")
    sid = lax.axis_index("s")
    _zero_acc(zb_v, acc_sh, sid)
    plsc.subcore_barrier()
    _mp_pipeline(h_hbm, ea3_hbm, src_hbm, srchi_hbm, dst_hbm, acc_sh,
                 srcv, dstv, (xg0, xg1), (pv0, pv1, pv2),
                 sems[0:2], sems[2:5], sems[5:7], sems[7:10], sems[10:13],
                 nch=_ET2 // _CB, ebase=sid * _ET2, lin_row=cid, cid=cid)
    plsc.subcore_barrier()
    _flush_acc(acc_sh, out_hbm, cid, sid)


def _b2(h_cat, src, src_hi, dst, ea_cat):
    return pl.kernel(
        _b2_body,
        out_type=jax.ShapeDtypeStruct((_NC, _NP, _D), jnp.float32),
        mesh=plsc.VectorSubcoreMesh(core_axis_name="c", subcore_axis_name="s"),
        scratch_types=_SC_SCRATCH,
    )(h_cat, src, src_hi, dst, ea_cat)


def _c1_body(x_ref, a_ref, w11_ref, b11_ref, w12_ref, b12_ref, h_ref):
    hin = x_ref[...] + a_ref[0] + a_ref[1]
    t = jnp.maximum(
        jnp.dot(hin, w11_ref[...], preferred_element_type=jnp.float32)
        + b11_ref[...], 0.0)
    h = jnp.maximum(
        jnp.dot(t, w12_ref[...], preferred_element_type=jnp.float32)
        + b12_ref[...], 0.0)
    h_ref[0] = h[:, :_D]
    h_ref[1] = h[:, _D:]


def _c1(x, a1, w11, b11, w12, b12):
    return pl.pallas_call(
        _c1_body,
        grid=(_NGB,),
        in_specs=[
            pl.BlockSpec((_NB, _D), lambda i: (i, 0)),
            pl.BlockSpec((2, _NB, _D), lambda i: (0, i, 0)),
            pl.BlockSpec((_D, _H), lambda i: (0, 0)),
            pl.BlockSpec((1, _H), lambda i: (0, 0)),
            pl.BlockSpec((_H, _H), lambda i: (0, 0)),
            pl.BlockSpec((1, _H), lambda i: (0, 0)),
        ],
        out_specs=pl.BlockSpec((2, _NB, _D), lambda i: (0, i, 0)),
        out_shape=jax.ShapeDtypeStruct((2, _N, _D), jnp.float32),
    )(x, a1, w11, b11, w12, b12)


def _c2_body(h_ref, a_ref, batch_ref, w21_ref, b21_ref, w22_ref, b22_ref,
             wout_ref, bout_ref, out_ref, acc_ref, cnt_ref):
    i = pl.program_id(0)
    h = jnp.concatenate([h_ref[0], h_ref[1]], axis=1)
    a2 = jnp.concatenate([a_ref[0], a_ref[1]], axis=1)
    t = jnp.maximum(
        jnp.dot(h + a2, w21_ref[...], preferred_element_type=jnp.float32)
        + b21_ref[...], 0.0)
    h2 = jnp.maximum(
        jnp.dot(t, w22_ref[...], preferred_element_type=jnp.float32)
        + b22_ref[...], 0.0)
    bb = batch_ref[0, 0, :]
    seg = lax.broadcasted_iota(jnp.int32, (_G, _NB), 0)
    oh = (bb[None, :] == seg).astype(jnp.float32)

    @pl.when(i == 0)
    def _():
        acc_ref[...] = jnp.zeros_like(acc_ref)
        cnt_ref[...] = jnp.zeros_like(cnt_ref)

    acc_ref[...] += jnp.dot(oh, h2, preferred_element_type=jnp.float32)
    cnt_ref[...] += jnp.broadcast_to(
        jnp.sum(oh, axis=1, keepdims=True), cnt_ref.shape)

    @pl.when(i == _NGB - 1)
    def _():
        pooled = acc_ref[...] / jnp.maximum(cnt_ref[:, 0:1], 1.0)
        out_ref[...] = (
            jnp.dot(pooled, wout_ref[...], preferred_element_type=jnp.float32)
            + bout_ref[...])


def _c2(h_cat, a2, batch_r, w21, b21, w22, b22, wout, bout):
    return pl.pallas_call(
        _c2_body,
        grid=(_NGB,),
        in_specs=[
            pl.BlockSpec((2, _NB, _D), lambda i: (0, i, 0)),
            pl.BlockSpec((2, _NB, _D), lambda i: (0, i, 0)),
            pl.BlockSpec((1, 1, _NB), lambda i: (i, 0, 0)),
            pl.BlockSpec((_H, _H), lambda i: (0, 0)),
            pl.BlockSpec((1, _H), lambda i: (0, 0)),
            pl.BlockSpec((_H, _H), lambda i: (0, 0)),
            pl.BlockSpec((1, _H), lambda i: (0, 0)),
            pl.BlockSpec((_H, _C), lambda i: (0, 0)),
            pl.BlockSpec((1, _C), lambda i: (0, 0)),
        ],
        out_specs=pl.BlockSpec((_G, _C), lambda i: (0, 0)),
        out_shape=jax.ShapeDtypeStruct((_G, _C), jnp.float32),
        scratch_shapes=[
            pltpu.VMEM((_G, _H), jnp.float32),
            pltpu.VMEM((_G, _D), jnp.float32),
        ],
    )(h_cat, a2, batch_r, w21, b21, w22, b22, wout, bout)


def kernel(x, edge_index, edge_attr, batch,
           W_e1, b_e1, W_e2, b_e2, W_c1e, b_c1e,
           W11, b11, W12, b12, W21, b21, W22, b22,
           W_out, b_out):
    src = edge_index[0]
    dst = edge_index[1]

    p1 = _enc_p1(edge_attr, W_e1, b_e1[None], W_e2, b_e2[None],
                 W_c1e, b_c1e[None])
    aggr1 = _b1(x, src, dst, p1.reshape(1, _E, _D))
    ea_cat = _enc_ea(edge_attr, W_e1, b_e1[None], W_e2, b_e2[None])
    h_cat = _c1(x, aggr1, W11, b11[None], W12, b12[None])
    aggr2 = _b2(h_cat.reshape(_NC * _N, _D), src, src + _N, dst, ea_cat)
    return _c2(h_cat, aggr2,
               batch.reshape(_NGB, 1, _NB),
               W21, b21[None], W22, b22[None], W_out, b_out[None])


# speedup vs baseline: 1.0738x; 1.0738x over previous
import functools

import jax
import jax.numpy as jnp
from jax import lax
from jax.experimental import pallas as pl
from jax.experimental.pallas import tpu as pltpu
from jax.experimental.pallas import tpu_sc as plsc

_N = 10000
_E = 320000
_D = 128
_DE = 16
_H = 256
_C = 10
_G = 64

_NC = 2
_NS = 16

_CB = 64
_EP = 321536
_EB = 2048
_NB = 1000
_NGB = _N // _NB
_ET1 = _EP // (_NC * _NS)
_ET2 = _EP // _NS
_NP = 10240
_ZR = _NP // _NS
_ZCH = 8


def _enc_p1_body(ea_ref, we1_ref, be1_ref, we2_ref, be2_ref, wc_ref, bc_ref,
                 p1_ref, wf_ref, bf_ref):
    @pl.when(pl.program_id(0) == 0)
    def _():
        wf_ref[...] = jnp.dot(we2_ref[...], wc_ref[...],
                              preferred_element_type=jnp.float32)
        bf_ref[...] = (
            jnp.dot(be2_ref[...], wc_ref[...],
                    preferred_element_type=jnp.float32) + bc_ref[...])

    t = jnp.maximum(
        jnp.dot(ea_ref[...], we1_ref[...], preferred_element_type=jnp.float32)
        + be1_ref[...], 0.0)
    p1_ref[...] = (
        jnp.dot(t, wf_ref[...], preferred_element_type=jnp.float32)
        + bf_ref[...])


def _enc_p1(edge_attr, we1, be1, we2, be2, wc, bc):
    return pl.pallas_call(
        _enc_p1_body,
        grid=(_EP // _EB,),
        in_specs=[
            pl.BlockSpec((_EB, _DE), lambda i: (i, 0)),
            pl.BlockSpec((_DE, _H), lambda i: (0, 0)),
            pl.BlockSpec((1, _H), lambda i: (0, 0)),
            pl.BlockSpec((_H, _H), lambda i: (0, 0)),
            pl.BlockSpec((1, _H), lambda i: (0, 0)),
            pl.BlockSpec((_H, _D), lambda i: (0, 0)),
            pl.BlockSpec((1, _D), lambda i: (0, 0)),
        ],
        out_specs=pl.BlockSpec((_EB, _D), lambda i: (i, 0)),
        out_shape=jax.ShapeDtypeStruct((_EP, _D), jnp.float32),
        scratch_shapes=[
            pltpu.VMEM((_H, _D), jnp.float32),
            pltpu.VMEM((1, _D), jnp.float32),
        ],
    )(edge_attr, we1, be1, we2, be2, wc, bc)


def _enc_ea_body(ea_ref, we1_ref, be1_ref, we2_ref, be2_ref, eac_ref):
    t = jnp.maximum(
        jnp.dot(ea_ref[...], we1_ref[...], preferred_element_type=jnp.float32)
        + be1_ref[...], 0.0)
    ea = jnp.dot(t, we2_ref[...], preferred_element_type=jnp.float32) + be2_ref[...]
    eac_ref[0] = ea[:, :_D]
    eac_ref[1] = ea[:, _D:]


def _enc_ea(edge_attr, we1, be1, we2, be2):
    return pl.pallas_call(
        _enc_ea_body,
        grid=(_EP // _EB,),
        in_specs=[
            pl.BlockSpec((_EB, _DE), lambda i: (i, 0)),
            pl.BlockSpec((_DE, _H), lambda i: (0, 0)),
            pl.BlockSpec((1, _H), lambda i: (0, 0)),
            pl.BlockSpec((_H, _H), lambda i: (0, 0)),
            pl.BlockSpec((1, _H), lambda i: (0, 0)),
        ],
        out_specs=pl.BlockSpec((2, _EB, _D), lambda i: (0, i, 0)),
        out_shape=jax.ShapeDtypeStruct((2, _EP, _D), jnp.float32),
    )(edge_attr, we1, be1, we2, be2)


def _zero_acc(zb_v, acc_sh, sid):
    @pl.loop(0, _ZCH)
    def _(j):
        for k in range(_D // 16):
            zb_v[j, pl.ds(k * 16, 16)] = jnp.zeros((16,), jnp.float32)

    @pl.loop(0, _ZR // _ZCH)
    def _(k):
        pltpu.sync_copy(zb_v, acc_sh.at[pl.ds(sid * _ZR + k * _ZCH, _ZCH)])


def _mp_pipeline(tab_hbm, lin3_hbm, srcA_hbm, srcB_hbm, dst_hbm, acc_sh,
                 srcv, dstv, xg, pv, sg, sl, ssrc, sdst, ss,
                 *, nch, ebase, lin_row, cid):
    def issue_idx(i, b2, b3):
        off = ebase + i * _CB

        @pl.when(cid == 0)
        def _():
            pltpu.async_copy(srcA_hbm.at[pl.ds(off, _CB)], srcv.at[b2],
                             ssrc[b2])

        @pl.when(cid == 1)
        def _():
            pltpu.async_copy(srcB_hbm.at[pl.ds(off, _CB)], srcv.at[b2],
                             ssrc[b2])

        pltpu.async_copy(dst_hbm.at[pl.ds(off, _CB)], dstv.at[b3], sdst[b3])

    def wait_src(b2):
        pltpu.make_async_copy(
            srcA_hbm.at[pl.ds(ebase, _CB)], srcv.at[b2], ssrc[b2]).wait()

    def wait_dst(b3):
        pltpu.make_async_copy(
            dst_hbm.at[pl.ds(ebase, _CB)], dstv.at[b3], sdst[b3]).wait()

    def issue_in(i, b2, b3):
        pltpu.async_copy(tab_hbm.at[srcv.at[b2]], xg[b2], sg[b2])
        pltpu.async_copy(lin3_hbm.at[lin_row, pl.ds(ebase + i * _CB, _CB)],
                         pv[b3], sl[b3])

    def wait_in(b2, b3):
        pltpu.make_async_copy(tab_hbm.at[srcv.at[b2]], xg[b2], sg[b2]).wait()
        pltpu.make_async_copy(lin3_hbm.at[lin_row, pl.ds(ebase, _CB)],
                              pv[b3], sl[b3]).wait()

    def compute(b2, b3):
        xb, pb = xg[b2], pv[b3]

        @pl.loop(0, _CB)
        def _(j):
            for k in range(_D // 16):
                sli = pl.ds(k * 16, 16)
                pb[j, sli] = jnp.maximum(xb[j, sli] + pb[j, sli], 0.0)

    def issue_sc(b3):
        pltpu.async_copy(pv[b3], acc_sh.at[dstv.at[b3]], ss[b3], add=True)

    def wait_ss(b3):
        pltpu.make_async_copy(pv[b3], acc_sh.at[dstv.at[b3]], ss[b3]).wait()

    def step(i, b2, b3, first=False, issue1=True, issue2=True):
        wait_in(b2, b3)
        if issue1:
            wait_src(1 - b2)
            issue_in(i + 1, 1 - b2, (b3 + 1) % 3)
        compute(b2, b3)
        wait_dst(b3)
        issue_sc(b3)
        if not first:
            wait_ss((b3 + 2) % 3)
        if issue2:
            issue_idx(i + 2, b2, (b3 + 2) % 3)

    issue_idx(0, 0, 0)
    issue_idx(1, 1, 1)
    wait_src(0)
    issue_in(0, 0, 0)
    step(0, 0, 0, first=True)
    step(1, 1, 1)

    steady = (nch - 4) // 6

    @pl.loop(0, steady)
    def _(s):
        i0 = 2 + s * 6
        for u in range(6):
            step(i0 + u, (2 + u) % 2, (2 + u) % 3)

    for i in range(2 + 6 * steady, nch):
        step(i, i % 2, i % 3, issue1=(i + 1 < nch), issue2=(i + 2 < nch))
    wait_ss((nch - 1) % 3)


def _flush_acc(acc_sh, out_hbm, cid, sid):
    row0 = sid * _ZR
    pltpu.sync_copy(acc_sh.at[pl.ds(row0, _ZR)],
                    out_hbm.at[cid, pl.ds(row0, _ZR)])


_SC_SCRATCH = [
    pltpu.VMEM((2, _CB), jnp.int32),
    pltpu.VMEM((3, _CB), jnp.int32),
    pltpu.VMEM((_CB, _D), jnp.float32),
    pltpu.VMEM((_CB, _D), jnp.float32),
    pltpu.VMEM((_CB, _D), jnp.float32),
    pltpu.VMEM((_CB, _D), jnp.float32),
    pltpu.VMEM((_CB, _D), jnp.float32),
    pltpu.VMEM((_ZCH, _D), jnp.float32),
    pltpu.VMEM_SHARED((_NP, _D), jnp.float32),
] + [pltpu.SemaphoreType.DMA] * 13


def _b1_body(x_hbm, src_hbm, dst_hbm, p13_hbm, out_hbm,
             srcv, dstv, xg0, xg1, pv0, pv1, pv2, zb_v, acc_sh, *sems):
    cid = lax.axis_index("c")
    sid = lax.axis_index("s")
    wid = cid * _NS + sid
    _zero_acc(zb_v, acc_sh, sid)
    plsc.subcore_barrier()
    _mp_pipeline(x_hbm, p13_hbm, src_hbm, src_hbm, dst_hbm, acc_sh,
                 srcv, dstv, (xg0, xg1), (pv0, pv1, pv2),
                 sems[0:2], sems[2:5], sems[5:7], sems[7:10], sems[10:13],
                 nch=_ET1 // _CB, ebase=wid * _ET1, lin_row=0, cid=cid)
    plsc.subcore_barrier()
    _flush_acc(acc_sh, out_hbm, cid, sid)


def _b1(x, src, dst, p13):
    return pl.kernel(
        _b1_body,
        out_type=jax.ShapeDtypeStruct((_NC, _NP, _D), jnp.float32),
        mesh=plsc.VectorSubcoreMesh(core_axis_name="c", subcore_axis_name="s"),
        scratch_types=_SC_SCRATCH,
    )(x, src, dst, p13)


def _b2_body(h_hbm, src_hbm, srchi_hbm, dst_hbm, ea3_hbm, out_hbm,
             srcv, dstv, xg0, xg1, pv0, pv1, pv2, zb_v, acc_sh, *sems):
    cid = lax.axis_index("c")
    sid = lax.axis_index("s")
    _zero_acc(zb_v, acc_sh, sid)
    plsc.subcore_barrier()
    _mp_pipeline(h_hbm, ea3_hbm, src_hbm, srchi_hbm, dst_hbm, acc_sh,
                 srcv, dstv, (xg0, xg1), (pv0, pv1, pv2),
                 sems[0:2], sems[2:5], sems[5:7], sems[7:10], sems[10:13],
                 nch=_ET2 // _CB, ebase=sid * _ET2, lin_row=cid, cid=cid)
    plsc.subcore_barrier()
    _flush_acc(acc_sh, out_hbm, cid, sid)


def _b2(h_cat, src, src_hi, dst, ea_cat):
    return pl.kernel(
        _b2_body,
        out_type=jax.ShapeDtypeStruct((_NC, _NP, _D), jnp.float32),
        mesh=plsc.VectorSubcoreMesh(core_axis_name="c", subcore_axis_name="s"),
        scratch_types=_SC_SCRATCH,
    )(h_cat, src, src_hi, dst, ea_cat)


def _c1_body(x_ref, a_ref, w11_ref, b11_ref, w12_ref, b12_ref, h_ref):
    hin = x_ref[...] + a_ref[0] + a_ref[1]
    t = jnp.maximum(
        jnp.dot(hin, w11_ref[...], preferred_element_type=jnp.float32)
        + b11_ref[...], 0.0)
    h = jnp.maximum(
        jnp.dot(t, w12_ref[...], preferred_element_type=jnp.float32)
        + b12_ref[...], 0.0)
    h_ref[0] = h[:, :_D]
    h_ref[1] = h[:, _D:]


def _c1(x, a1, w11, b11, w12, b12):
    return pl.pallas_call(
        _c1_body,
        grid=(_NGB,),
        in_specs=[
            pl.BlockSpec((_NB, _D), lambda i: (i, 0)),
            pl.BlockSpec((2, _NB, _D), lambda i: (0, i, 0)),
            pl.BlockSpec((_D, _H), lambda i: (0, 0)),
            pl.BlockSpec((1, _H), lambda i: (0, 0)),
            pl.BlockSpec((_H, _H), lambda i: (0, 0)),
            pl.BlockSpec((1, _H), lambda i: (0, 0)),
        ],
        out_specs=pl.BlockSpec((2, _NB, _D), lambda i: (0, i, 0)),
        out_shape=jax.ShapeDtypeStruct((2, _N, _D), jnp.float32),
    )(x, a1, w11, b11, w12, b12)


def _c2_body(h_ref, a_ref, batch_ref, w21_ref, b21_ref, w22_ref, b22_ref,
             wout_ref, bout_ref, out_ref, acc_ref, cnt_ref):
    i = pl.program_id(0)
    h = jnp.concatenate([h_ref[0], h_ref[1]], axis=1)
    a2 = jnp.concatenate([a_ref[0], a_ref[1]], axis=1)
    t = jnp.maximum(
        jnp.dot(h + a2, w21_ref[...], preferred_element_type=jnp.float32)
        + b21_ref[...], 0.0)
    h2 = jnp.maximum(
        jnp.dot(t, w22_ref[...], preferred_element_type=jnp.float32)
        + b22_ref[...], 0.0)
    bb = batch_ref[0, 0, :]
    seg = lax.broadcasted_iota(jnp.int32, (_G, _NB), 0)
    oh = (bb[None, :] == seg).astype(jnp.float32)

    @pl.when(i == 0)
    def _():
        acc_ref[...] = jnp.zeros_like(acc_ref)
        cnt_ref[...] = jnp.zeros_like(cnt_ref)

    acc_ref[...] += jnp.dot(oh, h2, preferred_element_type=jnp.float32)
    cnt_ref[...] += jnp.broadcast_to(
        jnp.sum(oh, axis=1, keepdims=True), cnt_ref.shape)

    @pl.when(i == _NGB - 1)
    def _():
        pooled = acc_ref[...] / jnp.maximum(cnt_ref[:, 0:1], 1.0)
        out_ref[...] = (
            jnp.dot(pooled, wout_ref[...], preferred_element_type=jnp.float32)
            + bout_ref[...])


def _c2(h_cat, a2, batch_r, w21, b21, w22, b22, wout, bout):
    return pl.pallas_call(
        _c2_body,
        grid=(_NGB,),
        in_specs=[
            pl.BlockSpec((2, _NB, _D), lambda i: (0, i, 0)),
            pl.BlockSpec((2, _NB, _D), lambda i: (0, i, 0)),
            pl.BlockSpec((1, 1, _NB), lambda i: (i, 0, 0)),
            pl.BlockSpec((_H, _H), lambda i: (0, 0)),
            pl.BlockSpec((1, _H), lambda i: (0, 0)),
            pl.BlockSpec((_H, _H), lambda i: (0, 0)),
            pl.BlockSpec((1, _H), lambda i: (0, 0)),
            pl.BlockSpec((_H, _C), lambda i: (0, 0)),
            pl.BlockSpec((1, _C), lambda i: (0, 0)),
        ],
        out_specs=pl.BlockSpec((_G, _C), lambda i: (0, 0)),
        out_shape=jax.ShapeDtypeStruct((_G, _C), jnp.float32),
        scratch_shapes=[
            pltpu.VMEM((_G, _H), jnp.float32),
            pltpu.VMEM((_G, _D), jnp.float32),
        ],
    )(h_cat, a2, batch_r, w21, b21, w22, b22, wout, bout)


def kernel(x, edge_index, edge_attr, batch,
           W_e1, b_e1, W_e2, b_e2, W_c1e, b_c1e,
           W11, b11, W12, b12, W21, b21, W22, b22,
           W_out, b_out):
    src = edge_index[0]
    dst = edge_index[1]

    pad = _EP - _E
    ea_pad = jnp.concatenate(
        [edge_attr, jnp.zeros((pad, _DE), jnp.float32)], axis=0)
    src_p = jnp.concatenate(
        [src, jnp.arange(pad, dtype=jnp.int32) % _N], axis=0)
    dst_p = jnp.concatenate(
        [dst, _N + (jnp.arange(pad, dtype=jnp.int32) % (_NP - _N))], axis=0)
    p1 = _enc_p1(ea_pad, W_e1, b_e1[None], W_e2, b_e2[None],
                 W_c1e, b_c1e[None])
    aggr1 = _b1(x, src_p, dst_p, p1.reshape(1, _EP, _D))
    ea_cat = _enc_ea(ea_pad, W_e1, b_e1[None], W_e2, b_e2[None])
    h_cat = _c1(x, aggr1, W11, b11[None], W12, b12[None])
    aggr2 = _b2(h_cat.reshape(_NC * _N, _D), src_p, src_p + _N, dst_p, ea_cat)
    return _c2(h_cat, aggr2,
               batch.reshape(_NGB, 1, _NB),
               W21, b21[None], W22, b22[None], W_out, b_out[None])


# speedup vs baseline: 1.0740x; 1.0002x over previous
import functools

import jax
import jax.numpy as jnp
from jax import lax
from jax.experimental import pallas as pl
from jax.experimental.pallas import tpu as pltpu
from jax.experimental.pallas import tpu_sc as plsc

_N = 10000
_E = 320000
_D = 128
_DE = 16
_H = 256
_C = 10
_G = 64

_NC = 2
_NS = 16

_CB = 64
_EP = 321536
_EB = 2048
_NB = 1000
_NGB = _N // _NB
_ET1 = _EP // (_NC * _NS)
_ET2 = _EP // _NS
_NP = 10240
_ZR = _NP // _NS
_ZCH = 8


def _enc_p1_body(ea_ref, we1_ref, be1_ref, we2_ref, be2_ref, wc_ref, bc_ref,
                 p1_ref, wf_ref, bf_ref):
    @pl.when(pl.program_id(0) == 0)
    def _():
        wf_ref[...] = jnp.dot(we2_ref[...], wc_ref[...],
                              preferred_element_type=jnp.float32)
        bf_ref[...] = (
            jnp.dot(be2_ref[...], wc_ref[...],
                    preferred_element_type=jnp.float32) + bc_ref[...])

    t = jnp.maximum(
        jnp.dot(ea_ref[...], we1_ref[...], preferred_element_type=jnp.float32)
        + be1_ref[...], 0.0)
    p1_ref[...] = (
        jnp.dot(t, wf_ref[...], preferred_element_type=jnp.float32)
        + bf_ref[...])


def _enc_p1(edge_attr, we1, be1, we2, be2, wc, bc):
    return pl.pallas_call(
        _enc_p1_body,
        grid=(_EP // _EB,),
        in_specs=[
            pl.BlockSpec((_EB, _DE), lambda i: (i, 0)),
            pl.BlockSpec((_DE, _H), lambda i: (0, 0)),
            pl.BlockSpec((1, _H), lambda i: (0, 0)),
            pl.BlockSpec((_H, _H), lambda i: (0, 0)),
            pl.BlockSpec((1, _H), lambda i: (0, 0)),
            pl.BlockSpec((_H, _D), lambda i: (0, 0)),
            pl.BlockSpec((1, _D), lambda i: (0, 0)),
        ],
        out_specs=pl.BlockSpec((_EB, _D), lambda i: (i, 0)),
        out_shape=jax.ShapeDtypeStruct((_EP, _D), jnp.float32),
        scratch_shapes=[
            pltpu.VMEM((_H, _D), jnp.float32),
            pltpu.VMEM((1, _D), jnp.float32),
        ],
    )(edge_attr, we1, be1, we2, be2, wc, bc)


def _enc_ea_body(ea_ref, we1_ref, be1_ref, we2_ref, be2_ref, eac_ref):
    t = jnp.maximum(
        jnp.dot(ea_ref[...], we1_ref[...], preferred_element_type=jnp.float32)
        + be1_ref[...], 0.0)
    ea = jnp.dot(t, we2_ref[...], preferred_element_type=jnp.float32) + be2_ref[...]
    eac_ref[0] = ea[:, :_D]
    eac_ref[1] = ea[:, _D:]


def _enc_ea(edge_attr, we1, be1, we2, be2):
    return pl.pallas_call(
        _enc_ea_body,
        grid=(_EP // _EB,),
        in_specs=[
            pl.BlockSpec((_EB, _DE), lambda i: (i, 0)),
            pl.BlockSpec((_DE, _H), lambda i: (0, 0)),
            pl.BlockSpec((1, _H), lambda i: (0, 0)),
            pl.BlockSpec((_H, _H), lambda i: (0, 0)),
            pl.BlockSpec((1, _H), lambda i: (0, 0)),
        ],
        out_specs=pl.BlockSpec((2, _EB, _D), lambda i: (0, i, 0)),
        out_shape=jax.ShapeDtypeStruct((2, _EP, _D), jnp.float32),
    )(edge_attr, we1, be1, we2, be2)


def _zero_acc(zb_v, acc_sh, sid):
    @pl.loop(0, _ZCH)
    def _(j):
        for k in range(_D // 16):
            zb_v[j, pl.ds(k * 16, 16)] = jnp.zeros((16,), jnp.float32)

    @pl.loop(0, _ZR // _ZCH)
    def _(k):
        pltpu.sync_copy(zb_v, acc_sh.at[pl.ds(sid * _ZR + k * _ZCH, _ZCH)])


def _mp_pipeline(tab_hbm, lin3_hbm, srcA_hbm, srcB_hbm, dst_hbm, acc_sh,
                 srcv, dstv, xg, pv, sg, sl, ssrc, sdst, ss,
                 *, nch, ebase, lin_row, cid):
    def issue_idx(i, b2, b3):
        off = ebase + i * _CB

        if srcB_hbm is None:
            pltpu.async_copy(srcA_hbm.at[pl.ds(off, _CB)], srcv.at[b2],
                             ssrc[b2])
        else:
            @pl.when(cid == 0)
            def _():
                pltpu.async_copy(srcA_hbm.at[pl.ds(off, _CB)], srcv.at[b2],
                                 ssrc[b2])

            @pl.when(cid == 1)
            def _():
                pltpu.async_copy(srcB_hbm.at[pl.ds(off, _CB)], srcv.at[b2],
                                 ssrc[b2])

        pltpu.async_copy(dst_hbm.at[pl.ds(off, _CB)], dstv.at[b3], sdst[b3])

    def wait_src(b2):
        pltpu.make_async_copy(
            srcA_hbm.at[pl.ds(ebase, _CB)], srcv.at[b2], ssrc[b2]).wait()

    def wait_dst(b3):
        pltpu.make_async_copy(
            dst_hbm.at[pl.ds(ebase, _CB)], dstv.at[b3], sdst[b3]).wait()

    def issue_in(i, b2, b3):
        pltpu.async_copy(tab_hbm.at[srcv.at[b2]], xg[b2], sg[b2])
        pltpu.async_copy(lin3_hbm.at[lin_row, pl.ds(ebase + i * _CB, _CB)],
                         pv[b3], sl[b3])

    def wait_in(b2, b3):
        pltpu.make_async_copy(tab_hbm.at[srcv.at[b2]], xg[b2], sg[b2]).wait()
        pltpu.make_async_copy(lin3_hbm.at[lin_row, pl.ds(ebase, _CB)],
                              pv[b3], sl[b3]).wait()

    def compute(b2, b3):
        xb, pb = xg[b2], pv[b3]

        @pl.loop(0, _CB)
        def _(j):
            for k in range(_D // 16):
                sli = pl.ds(k * 16, 16)
                pb[j, sli] = jnp.maximum(xb[j, sli] + pb[j, sli], 0.0)

    def issue_sc(b3):
        pltpu.async_copy(pv[b3], acc_sh.at[dstv.at[b3]], ss[b3], add=True)

    def wait_ss(b3):
        pltpu.make_async_copy(pv[b3], acc_sh.at[dstv.at[b3]], ss[b3]).wait()

    def step(i, b2, b3, first=False, issue1=True, issue2=True):
        wait_in(b2, b3)
        if issue1:
            wait_src(1 - b2)
            issue_in(i + 1, 1 - b2, (b3 + 1) % 3)
        compute(b2, b3)
        wait_dst(b3)
        issue_sc(b3)
        if not first:
            wait_ss((b3 + 2) % 3)
        if issue2:
            issue_idx(i + 2, b2, (b3 + 2) % 3)

    issue_idx(0, 0, 0)
    issue_idx(1, 1, 1)
    wait_src(0)
    issue_in(0, 0, 0)
    step(0, 0, 0, first=True)
    step(1, 1, 1)

    steady = (nch - 4) // 6

    @pl.loop(0, steady)
    def _(s):
        i0 = 2 + s * 6
        for u in range(6):
            step(i0 + u, (2 + u) % 2, (2 + u) % 3)

    for i in range(2 + 6 * steady, nch):
        step(i, i % 2, i % 3, issue1=(i + 1 < nch), issue2=(i + 2 < nch))
    wait_ss((nch - 1) % 3)


def _flush_acc(acc_sh, out_hbm, cid, sid):
    row0 = sid * _ZR
    pltpu.sync_copy(acc_sh.at[pl.ds(row0, _ZR)],
                    out_hbm.at[cid, pl.ds(row0, _ZR)])


_SC_SCRATCH = [
    pltpu.VMEM((2, _CB), jnp.int32),
    pltpu.VMEM((3, _CB), jnp.int32),
    pltpu.VMEM((_CB, _D), jnp.float32),
    pltpu.VMEM((_CB, _D), jnp.float32),
    pltpu.VMEM((_CB, _D), jnp.float32),
    pltpu.VMEM((_CB, _D), jnp.float32),
    pltpu.VMEM((_CB, _D), jnp.float32),
    pltpu.VMEM((_ZCH, _D), jnp.float32),
    pltpu.VMEM_SHARED((_NP, _D), jnp.float32),
] + [pltpu.SemaphoreType.DMA] * 13


def _b1_body(x_hbm, src_hbm, dst_hbm, p13_hbm, out_hbm,
             srcv, dstv, xg0, xg1, pv0, pv1, pv2, zb_v, acc_sh, *sems):
    cid = lax.axis_index("c")
    sid = lax.axis_index("s")
    wid = cid * _NS + sid
    _zero_acc(zb_v, acc_sh, sid)
    plsc.subcore_barrier()
    _mp_pipeline(x_hbm, p13_hbm, src_hbm, None, dst_hbm, acc_sh,
                 srcv, dstv, (xg0, xg1), (pv0, pv1, pv2),
                 sems[0:2], sems[2:5], sems[5:7], sems[7:10], sems[10:13],
                 nch=_ET1 // _CB, ebase=wid * _ET1, lin_row=0, cid=cid)
    plsc.subcore_barrier()
    _flush_acc(acc_sh, out_hbm, cid, sid)


def _b1(x, src, dst, p13):
    return pl.kernel(
        _b1_body,
        out_type=jax.ShapeDtypeStruct((_NC, _NP, _D), jnp.float32),
        mesh=plsc.VectorSubcoreMesh(core_axis_name="c", subcore_axis_name="s"),
        scratch_types=_SC_SCRATCH,
    )(x, src, dst, p13)


def _b2_body(h_hbm, src_hbm, srchi_hbm, dst_hbm, ea3_hbm, out_hbm,
             srcv, dstv, xg0, xg1, pv0, pv1, pv2, zb_v, acc_sh, *sems):
    cid = lax.axis_index("c")
    sid = lax.axis_index("s")
    _zero_acc(zb_v, acc_sh, sid)
    plsc.subcore_barrier()
    _mp_pipeline(h_hbm, ea3_hbm, src_hbm, srchi_hbm, dst_hbm, acc_sh,
                 srcv, dstv, (xg0, xg1), (pv0, pv1, pv2),
                 sems[0:2], sems[2:5], sems[5:7], sems[7:10], sems[10:13],
                 nch=_ET2 // _CB, ebase=sid * _ET2, lin_row=cid, cid=cid)
    plsc.subcore_barrier()
    _flush_acc(acc_sh, out_hbm, cid, sid)


def _b2(h_cat, src, src_hi, dst, ea_cat):
    return pl.kernel(
        _b2_body,
        out_type=jax.ShapeDtypeStruct((_NC, _NP, _D), jnp.float32),
        mesh=plsc.VectorSubcoreMesh(core_axis_name="c", subcore_axis_name="s"),
        scratch_types=_SC_SCRATCH,
    )(h_cat, src, src_hi, dst, ea_cat)


def _c1_body(x_ref, a_ref, w11_ref, b11_ref, w12_ref, b12_ref, h_ref):
    hin = x_ref[...] + a_ref[0] + a_ref[1]
    t = jnp.maximum(
        jnp.dot(hin, w11_ref[...], preferred_element_type=jnp.float32)
        + b11_ref[...], 0.0)
    h = jnp.maximum(
        jnp.dot(t, w12_ref[...], preferred_element_type=jnp.float32)
        + b12_ref[...], 0.0)
    h_ref[0] = h[:, :_D]
    h_ref[1] = h[:, _D:]


def _c1(x, a1, w11, b11, w12, b12):
    return pl.pallas_call(
        _c1_body,
        grid=(_NGB,),
        in_specs=[
            pl.BlockSpec((_NB, _D), lambda i: (i, 0)),
            pl.BlockSpec((2, _NB, _D), lambda i: (0, i, 0)),
            pl.BlockSpec((_D, _H), lambda i: (0, 0)),
            pl.BlockSpec((1, _H), lambda i: (0, 0)),
            pl.BlockSpec((_H, _H), lambda i: (0, 0)),
            pl.BlockSpec((1, _H), lambda i: (0, 0)),
        ],
        out_specs=pl.BlockSpec((2, _NB, _D), lambda i: (0, i, 0)),
        out_shape=jax.ShapeDtypeStruct((2, _N, _D), jnp.float32),
    )(x, a1, w11, b11, w12, b12)


def _c2_body(h_ref, a_ref, batch_ref, w21_ref, b21_ref, w22_ref, b22_ref,
             wout_ref, bout_ref, out_ref, acc_ref, cnt_ref):
    i = pl.program_id(0)
    h = jnp.concatenate([h_ref[0], h_ref[1]], axis=1)
    a2 = jnp.concatenate([a_ref[0], a_ref[1]], axis=1)
    t = jnp.maximum(
        jnp.dot(h + a2, w21_ref[...], preferred_element_type=jnp.float32)
        + b21_ref[...], 0.0)
    h2 = jnp.maximum(
        jnp.dot(t, w22_ref[...], preferred_element_type=jnp.float32)
        + b22_ref[...], 0.0)
    bb = batch_ref[0, 0, :]
    seg = lax.broadcasted_iota(jnp.int32, (_G, _NB), 0)
    oh = (bb[None, :] == seg).astype(jnp.float32)

    @pl.when(i == 0)
    def _():
        acc_ref[...] = jnp.zeros_like(acc_ref)
        cnt_ref[...] = jnp.zeros_like(cnt_ref)

    acc_ref[...] += jnp.dot(oh, h2, preferred_element_type=jnp.float32)
    cnt_ref[...] += jnp.broadcast_to(
        jnp.sum(oh, axis=1, keepdims=True), cnt_ref.shape)

    @pl.when(i == _NGB - 1)
    def _():
        pooled = acc_ref[...] / jnp.maximum(cnt_ref[:, 0:1], 1.0)
        out_ref[...] = (
            jnp.dot(pooled, wout_ref[...], preferred_element_type=jnp.float32)
            + bout_ref[...])


def _c2(h_cat, a2, batch_r, w21, b21, w22, b22, wout, bout):
    return pl.pallas_call(
        _c2_body,
        grid=(_NGB,),
        in_specs=[
            pl.BlockSpec((2, _NB, _D), lambda i: (0, i, 0)),
            pl.BlockSpec((2, _NB, _D), lambda i: (0, i, 0)),
            pl.BlockSpec((1, 1, _NB), lambda i: (i, 0, 0)),
            pl.BlockSpec((_H, _H), lambda i: (0, 0)),
            pl.BlockSpec((1, _H), lambda i: (0, 0)),
            pl.BlockSpec((_H, _H), lambda i: (0, 0)),
            pl.BlockSpec((1, _H), lambda i: (0, 0)),
            pl.BlockSpec((_H, _C), lambda i: (0, 0)),
            pl.BlockSpec((1, _C), lambda i: (0, 0)),
        ],
        out_specs=pl.BlockSpec((_G, _C), lambda i: (0, 0)),
        out_shape=jax.ShapeDtypeStruct((_G, _C), jnp.float32),
        scratch_shapes=[
            pltpu.VMEM((_G, _H), jnp.float32),
            pltpu.VMEM((_G, _D), jnp.float32),
        ],
    )(h_cat, a2, batch_r, w21, b21, w22, b22, wout, bout)


def kernel(x, edge_index, edge_attr, batch,
           W_e1, b_e1, W_e2, b_e2, W_c1e, b_c1e,
           W11, b11, W12, b12, W21, b21, W22, b22,
           W_out, b_out):
    src = edge_index[0]
    dst = edge_index[1]

    pad = _EP - _E
    ea_pad = jnp.concatenate(
        [edge_attr, jnp.zeros((pad, _DE), jnp.float32)], axis=0)
    src_p = jnp.concatenate(
        [src, jnp.arange(pad, dtype=jnp.int32) % _N], axis=0)
    dst_p = jnp.concatenate(
        [dst, _N + (jnp.arange(pad, dtype=jnp.int32) % (_NP - _N))], axis=0)
    p1 = _enc_p1(ea_pad, W_e1, b_e1[None], W_e2, b_e2[None],
                 W_c1e, b_c1e[None])
    ea_cat = _enc_ea(ea_pad, W_e1, b_e1[None], W_e2, b_e2[None])
    aggr1 = _b1(x, src_p, dst_p, p1.reshape(1, _EP, _D))
    h_cat = _c1(x, aggr1, W11, b11[None], W12, b12[None])
    aggr2 = _b2(h_cat.reshape(_NC * _N, _D), src_p, src_p + _N, dst_p, ea_cat)
    return _c2(h_cat, aggr2,
               batch.reshape(_NGB, 1, _NB),
               W21, b21[None], W22, b22[None], W_out, b_out[None])


# speedup vs baseline: 1.1085x; 1.0321x over previous
import functools

import jax
import jax.numpy as jnp
from jax import lax
from jax.experimental import pallas as pl
from jax.experimental.pallas import tpu as pltpu
from jax.experimental.pallas import tpu_sc as plsc

_N = 10000
_E = 320000
_D = 128
_DE = 16
_H = 256
_C = 10
_G = 64

_NC = 2
_NS = 16

_CB = 64
_EP = 321536
_EB = 2048
_NB = 1000
_NGB = _N // _NB
_ET1 = _EP // (_NC * _NS)
_ET2 = _EP // _NS
_NP = 10240
_ZR = _NP // _NS
_ZCH = 8


def _enc_p1_body(ea_ref, we1_ref, be1_ref, we2_ref, be2_ref, wc_ref, bc_ref,
                 p1_ref, wf_ref, bf_ref):
    @pl.when(pl.program_id(0) == 0)
    def _():
        wf_ref[...] = jnp.dot(we2_ref[...], wc_ref[...],
                              preferred_element_type=jnp.float32)
        bf_ref[...] = (
            jnp.dot(be2_ref[...], wc_ref[...],
                    preferred_element_type=jnp.float32) + bc_ref[...])

    t = jnp.maximum(
        jnp.dot(ea_ref[...], we1_ref[...], preferred_element_type=jnp.float32)
        + be1_ref[...], 0.0)
    p1_ref[...] = (
        jnp.dot(t, wf_ref[...], preferred_element_type=jnp.float32)
        + bf_ref[...])


def _enc_p1(edge_attr, we1, be1, we2, be2, wc, bc):
    return pl.pallas_call(
        _enc_p1_body,
        grid=(_EP // _EB,),
        in_specs=[
            pl.BlockSpec((_EB, _DE), lambda i: (i, 0)),
            pl.BlockSpec((_DE, _H), lambda i: (0, 0)),
            pl.BlockSpec((1, _H), lambda i: (0, 0)),
            pl.BlockSpec((_H, _H), lambda i: (0, 0)),
            pl.BlockSpec((1, _H), lambda i: (0, 0)),
            pl.BlockSpec((_H, _D), lambda i: (0, 0)),
            pl.BlockSpec((1, _D), lambda i: (0, 0)),
        ],
        out_specs=pl.BlockSpec((_EB, _D), lambda i: (i, 0)),
        out_shape=jax.ShapeDtypeStruct((_EP, _D), jnp.float32),
        scratch_shapes=[
            pltpu.VMEM((_H, _D), jnp.float32),
            pltpu.VMEM((1, _D), jnp.float32),
        ],
    )(edge_attr, we1, be1, we2, be2, wc, bc)


def _enc_ea_body(ea_ref, we1_ref, be1_ref, we2_ref, be2_ref, eac_ref):
    t = jnp.maximum(
        jnp.dot(ea_ref[...], we1_ref[...], preferred_element_type=jnp.float32)
        + be1_ref[...], 0.0)
    ea = jnp.dot(t, we2_ref[...], preferred_element_type=jnp.float32) + be2_ref[...]
    eac_ref[0] = ea[:, :_D]
    eac_ref[1] = ea[:, _D:]


def _enc_ea(edge_attr, we1, be1, we2, be2):
    return pl.pallas_call(
        _enc_ea_body,
        grid=(_EP // _EB,),
        in_specs=[
            pl.BlockSpec((_EB, _DE), lambda i: (i, 0)),
            pl.BlockSpec((_DE, _H), lambda i: (0, 0)),
            pl.BlockSpec((1, _H), lambda i: (0, 0)),
            pl.BlockSpec((_H, _H), lambda i: (0, 0)),
            pl.BlockSpec((1, _H), lambda i: (0, 0)),
        ],
        out_specs=pl.BlockSpec((2, _EB, _D), lambda i: (0, i, 0)),
        out_shape=jax.ShapeDtypeStruct((2, _EP, _D), jnp.float32),
    )(edge_attr, we1, be1, we2, be2)


def _zero_acc(zb_v, acc_sh, sid):
    @pl.loop(0, _ZCH)
    def _(j):
        for k in range(_D // 16):
            zb_v[j, pl.ds(k * 16, 16)] = jnp.zeros((16,), jnp.float32)

    @pl.loop(0, _ZR // _ZCH)
    def _(k):
        pltpu.sync_copy(zb_v, acc_sh.at[pl.ds(sid * _ZR + k * _ZCH, _ZCH)])


def _mp_pipeline(tab_hbm, lin3_hbm, srcA_hbm, srcB_hbm, dst_hbm, acc_sh,
                 srcv, dstv, xg, pv, sg, sl, ssrc, sdst, ss,
                 *, nch, ebase, lin_row, cid):
    def issue_idx(i, b2, b3):
        off = ebase + i * _CB

        if srcB_hbm is None:
            pltpu.async_copy(srcA_hbm.at[pl.ds(off, _CB)], srcv.at[b2],
                             ssrc[b2])
        else:
            @pl.when(cid == 0)
            def _():
                pltpu.async_copy(srcA_hbm.at[pl.ds(off, _CB)], srcv.at[b2],
                                 ssrc[b2])

            @pl.when(cid == 1)
            def _():
                pltpu.async_copy(srcB_hbm.at[pl.ds(off, _CB)], srcv.at[b2],
                                 ssrc[b2])

        pltpu.async_copy(dst_hbm.at[pl.ds(off, _CB)], dstv.at[b3], sdst[b3])

    def wait_src(b2):
        pltpu.make_async_copy(
            srcA_hbm.at[pl.ds(ebase, _CB)], srcv.at[b2], ssrc[b2]).wait()

    def wait_dst(b3):
        pltpu.make_async_copy(
            dst_hbm.at[pl.ds(ebase, _CB)], dstv.at[b3], sdst[b3]).wait()

    def issue_in(i, b2, b3):
        pltpu.async_copy(tab_hbm.at[srcv.at[b2]], xg[b2], sg[b2])
        pltpu.async_copy(lin3_hbm.at[lin_row, pl.ds(ebase + i * _CB, _CB)],
                         pv[b3], sl[b3])

    def wait_in(b2, b3):
        pltpu.make_async_copy(tab_hbm.at[srcv.at[b2]], xg[b2], sg[b2]).wait()
        pltpu.make_async_copy(lin3_hbm.at[lin_row, pl.ds(ebase, _CB)],
                              pv[b3], sl[b3]).wait()

    def compute(b2, b3):
        xb, pb = xg[b2], pv[b3]

        @pl.loop(0, _CB)
        def _(j):
            for k in range(_D // 16):
                sli = pl.ds(k * 16, 16)
                pb[j, sli] = jnp.maximum(xb[j, sli] + pb[j, sli], 0.0)

    def issue_sc(b3):
        pltpu.async_copy(pv[b3], acc_sh.at[dstv.at[b3]], ss[b3], add=True)

    def wait_ss(b3):
        pltpu.make_async_copy(pv[b3], acc_sh.at[dstv.at[b3]], ss[b3]).wait()

    def step(i, b2, b3, first=False, issue1=True, issue2=True):
        wait_in(b2, b3)
        if issue1:
            wait_src(1 - b2)
            issue_in(i + 1, 1 - b2, (b3 + 1) % 3)
        compute(b2, b3)
        wait_dst(b3)
        issue_sc(b3)
        if not first:
            wait_ss((b3 + 2) % 3)
        if issue2:
            issue_idx(i + 2, b2, (b3 + 2) % 3)

    issue_idx(0, 0, 0)
    issue_idx(1, 1, 1)
    wait_src(0)
    issue_in(0, 0, 0)
    step(0, 0, 0, first=True)
    step(1, 1, 1)

    steady = (nch - 4) // 6

    @pl.loop(0, steady)
    def _(s):
        i0 = 2 + s * 6
        for u in range(6):
            step(i0 + u, (2 + u) % 2, (2 + u) % 3)

    for i in range(2 + 6 * steady, nch):
        step(i, i % 2, i % 3, issue1=(i + 1 < nch), issue2=(i + 2 < nch))
    wait_ss((nch - 1) % 3)


def _flush_acc(acc_sh, out_hbm, cid, sid):
    row0 = sid * _ZR
    pltpu.sync_copy(acc_sh.at[pl.ds(row0, _ZR)],
                    out_hbm.at[cid, pl.ds(row0, _ZR)])


_SC_SCRATCH = [
    pltpu.VMEM((2, _CB), jnp.int32),
    pltpu.VMEM((3, _CB), jnp.int32),
    pltpu.VMEM((_CB, _D), jnp.float32),
    pltpu.VMEM((_CB, _D), jnp.float32),
    pltpu.VMEM((_CB, _D), jnp.float32),
    pltpu.VMEM((_CB, _D), jnp.float32),
    pltpu.VMEM((_CB, _D), jnp.float32),
    pltpu.VMEM((_ZCH, _D), jnp.float32),
    pltpu.VMEM_SHARED((_NP, _D), jnp.float32),
] + [pltpu.SemaphoreType.DMA] * 13


def _b1_body(x_hbm, src_hbm, dst_hbm, p13_hbm, out_hbm,
             srcv, dstv, xg0, xg1, pv0, pv1, pv2, zb_v, acc_sh, *sems):
    cid = lax.axis_index("c")
    sid = lax.axis_index("s")
    wid = cid * _NS + sid
    _zero_acc(zb_v, acc_sh, sid)
    plsc.subcore_barrier()
    _mp_pipeline(x_hbm, p13_hbm, src_hbm, None, dst_hbm, acc_sh,
                 srcv, dstv, (xg0, xg1), (pv0, pv1, pv2),
                 sems[0:2], sems[2:5], sems[5:7], sems[7:10], sems[10:13],
                 nch=_ET1 // _CB, ebase=wid * _ET1, lin_row=0, cid=cid)
    plsc.subcore_barrier()
    _flush_acc(acc_sh, out_hbm, cid, sid)


def _b1(x, src, dst, p13):
    return pl.kernel(
        _b1_body,
        out_type=jax.ShapeDtypeStruct((_NC, _NP, _D), jnp.float32),
        mesh=plsc.VectorSubcoreMesh(core_axis_name="c", subcore_axis_name="s"),
        scratch_types=_SC_SCRATCH,
    )(x, src, dst, p13)


def _b2_body(h_hbm, src_hbm, srchi_hbm, dst_hbm, ea3_hbm, out_hbm,
             srcv, dstv, xg0, xg1, pv0, pv1, pv2, zb_v, acc_sh, *sems):
    cid = lax.axis_index("c")
    sid = lax.axis_index("s")
    _zero_acc(zb_v, acc_sh, sid)
    plsc.subcore_barrier()
    _mp_pipeline(h_hbm, ea3_hbm, src_hbm, srchi_hbm, dst_hbm, acc_sh,
                 srcv, dstv, (xg0, xg1), (pv0, pv1, pv2),
                 sems[0:2], sems[2:5], sems[5:7], sems[7:10], sems[10:13],
                 nch=_ET2 // _CB, ebase=sid * _ET2, lin_row=cid, cid=cid)
    plsc.subcore_barrier()
    _flush_acc(acc_sh, out_hbm, cid, sid)


def _b2(h_cat, src, src_hi, dst, ea_cat):
    return pl.kernel(
        _b2_body,
        out_type=jax.ShapeDtypeStruct((_NC, _NP, _D), jnp.float32),
        mesh=plsc.VectorSubcoreMesh(core_axis_name="c", subcore_axis_name="s"),
        scratch_types=_SC_SCRATCH,
    )(h_cat, src, src_hi, dst, ea_cat)


def _c1_body(x_ref, a_ref, w11_ref, b11_ref, w12_ref, b12_ref, h_ref):
    hin = x_ref[...] + a_ref[0] + a_ref[1]
    t = jnp.maximum(
        jnp.dot(hin, w11_ref[...], preferred_element_type=jnp.float32)
        + b11_ref[...], 0.0)
    h = jnp.maximum(
        jnp.dot(t, w12_ref[...], preferred_element_type=jnp.float32)
        + b12_ref[...], 0.0)
    h_ref[0] = h[:, :_D]
    h_ref[1] = h[:, _D:]


def _c1(x, a1, w11, b11, w12, b12):
    return pl.pallas_call(
        _c1_body,
        grid=(_NGB,),
        in_specs=[
            pl.BlockSpec((_NB, _D), lambda i: (i, 0)),
            pl.BlockSpec((2, _NB, _D), lambda i: (0, i, 0)),
            pl.BlockSpec((_D, _H), lambda i: (0, 0)),
            pl.BlockSpec((1, _H), lambda i: (0, 0)),
            pl.BlockSpec((_H, _H), lambda i: (0, 0)),
            pl.BlockSpec((1, _H), lambda i: (0, 0)),
        ],
        out_specs=pl.BlockSpec((2, _NB, _D), lambda i: (0, i, 0)),
        out_shape=jax.ShapeDtypeStruct((2, _N, _D), jnp.float32),
    )(x, a1, w11, b11, w12, b12)


def _c2_body(h_ref, a_ref, batch_ref, w21_ref, b21_ref, w22_ref, b22_ref,
             wout_ref, bout_ref, out_ref, acc_ref, cnt_ref):
    i = pl.program_id(0)
    h = jnp.concatenate([h_ref[0], h_ref[1]], axis=1)
    a2 = jnp.concatenate([a_ref[0], a_ref[1]], axis=1)
    t = jnp.maximum(
        jnp.dot(h + a2, w21_ref[...], preferred_element_type=jnp.float32)
        + b21_ref[...], 0.0)
    h2 = jnp.maximum(
        jnp.dot(t, w22_ref[...], preferred_element_type=jnp.float32)
        + b22_ref[...], 0.0)
    bb = batch_ref[0, 0, :]
    seg = lax.broadcasted_iota(jnp.int32, (_G, _NB), 0)
    oh = (bb[None, :] == seg).astype(jnp.float32)

    @pl.when(i == 0)
    def _():
        acc_ref[...] = jnp.zeros_like(acc_ref)
        cnt_ref[...] = jnp.zeros_like(cnt_ref)

    acc_ref[...] += jnp.dot(oh, h2, preferred_element_type=jnp.float32)
    cnt_ref[...] += jnp.broadcast_to(
        jnp.sum(oh, axis=1, keepdims=True), cnt_ref.shape)

    @pl.when(i == _NGB - 1)
    def _():
        pooled = acc_ref[...] / jnp.maximum(cnt_ref[:, 0:1], 1.0)
        out_ref[...] = (
            jnp.dot(pooled, wout_ref[...], preferred_element_type=jnp.float32)
            + bout_ref[...])


def _c2(h_cat, a2, batch_r, w21, b21, w22, b22, wout, bout):
    return pl.pallas_call(
        _c2_body,
        grid=(_NGB,),
        in_specs=[
            pl.BlockSpec((2, _NB, _D), lambda i: (0, i, 0)),
            pl.BlockSpec((2, _NB, _D), lambda i: (0, i, 0)),
            pl.BlockSpec((1, 1, _NB), lambda i: (i, 0, 0)),
            pl.BlockSpec((_H, _H), lambda i: (0, 0)),
            pl.BlockSpec((1, _H), lambda i: (0, 0)),
            pl.BlockSpec((_H, _H), lambda i: (0, 0)),
            pl.BlockSpec((1, _H), lambda i: (0, 0)),
            pl.BlockSpec((_H, _C), lambda i: (0, 0)),
            pl.BlockSpec((1, _C), lambda i: (0, 0)),
        ],
        out_specs=pl.BlockSpec((_G, _C), lambda i: (0, 0)),
        out_shape=jax.ShapeDtypeStruct((_G, _C), jnp.float32),
        scratch_shapes=[
            pltpu.VMEM((_G, _H), jnp.float32),
            pltpu.VMEM((_G, _D), jnp.float32),
        ],
    )(h_cat, a2, batch_r, w21, b21, w22, b22, wout, bout)


def kernel(x, edge_index, edge_attr, batch,
           W_e1, b_e1, W_e2, b_e2, W_c1e, b_c1e,
           W11, b11, W12, b12, W21, b21, W22, b22,
           W_out, b_out):
    src = edge_index[0]
    dst = edge_index[1]

    pad = _EP - _E
    src_p = jnp.concatenate(
        [src, jnp.arange(pad, dtype=jnp.int32) % _N], axis=0)
    dst_p = jnp.concatenate(
        [dst, _N + (jnp.arange(pad, dtype=jnp.int32) % (_NP - _N))], axis=0)
    p1 = _enc_p1(edge_attr, W_e1, b_e1[None], W_e2, b_e2[None],
                 W_c1e, b_c1e[None])
    ea_cat = _enc_ea(edge_attr, W_e1, b_e1[None], W_e2, b_e2[None])
    aggr1 = _b1(x, src_p, dst_p, p1.reshape(1, _EP, _D))
    h_cat = _c1(x, aggr1, W11, b11[None], W12, b12[None])
    aggr2 = _b2(h_cat.reshape(_NC * _N, _D), src_p, src_p + _N, dst_p, ea_cat)
    return _c2(h_cat, aggr2,
               batch.reshape(_NGB, 1, _NB),
               W21, b21[None], W22, b22[None], W_out, b_out[None])
